# speedup vs baseline: 1.6087x; 1.6087x over previous
import functools

import jax
import jax.numpy as jnp
from jax.experimental import pallas as pl
from jax.experimental.pallas import tpu as pltpu

T = 2048
H = 1024
F = 2048
E = 8
SOFTCAP = 30.0

TT = 256
T_TILES = T // TT


def _router_body(x_ref, wg_ref, comb_ref):
    x = x_ref[...]
    logits = jax.lax.dot_general(
        x, wg_ref[...], (((1,), (1,)), ((), ())),
        preferred_element_type=jnp.float32)
    logits = SOFTCAP * jnp.tanh(logits / SOFTCAP)
    m = jnp.max(logits, axis=-1, keepdims=True)
    p = jnp.exp(logits - m)
    p = p / jnp.sum(p, axis=-1, keepdims=True)
    e_iota = jax.lax.broadcasted_iota(jnp.int32, (T, E), 1)
    big = jnp.int32(E)
    m1 = jnp.max(p, axis=-1, keepdims=True)
    i1 = jnp.min(jnp.where(p == m1, e_iota, big), axis=-1, keepdims=True)
    sel1 = e_iota == i1
    p2 = jnp.where(sel1, -jnp.inf, p)
    m2 = jnp.max(p2, axis=-1, keepdims=True)
    i2 = jnp.min(jnp.where(p2 == m2, e_iota, big), axis=-1, keepdims=True)
    sel2 = e_iota == i2
    comb_ref[...] = jnp.where(sel1 | sel2, p, 0.0)


def _moe_body(x_ref, comb_ref, w1_ref, w3_ref, w2_ref, out_ref):
    e = pl.program_id(0)
    t = pl.program_id(1)
    x = x_ref[...]
    h1 = jax.lax.dot_general(
        x, w1_ref[0], (((1,), (1,)), ((), ())),
        preferred_element_type=jnp.float32)
    h3 = jax.lax.dot_general(
        x, w3_ref[0], (((1,), (1,)), ((), ())),
        preferred_element_type=jnp.float32)
    act = jax.lax.erf(h1 * 0.7071067811865476)
    act = (0.5 * h1) * (1.0 + act) * h3
    y = jax.lax.dot_general(
        act.astype(jnp.bfloat16), w2_ref[0], (((1,), (1,)), ((), ())),
        preferred_element_type=jnp.float32)
    onehot = (jax.lax.broadcasted_iota(jnp.int32, (E, 1), 0) == e
              ).astype(jnp.float32)
    c_col = jax.lax.dot_general(
        comb_ref[...], onehot, (((1,), (0,)), ((), ())),
        preferred_element_type=jnp.float32)
    contrib = c_col * y

    @pl.when(e == 0)
    def _init():
        out_ref[pl.ds(t * TT, TT), :] = contrib

    @pl.when(e != 0)
    def _acc():
        out_ref[pl.ds(t * TT, TT), :] += contrib


@jax.jit
def kernel(hidden_states, w_gate, w1, w3, w2):
    combine = pl.pallas_call(
        _router_body,
        out_shape=jax.ShapeDtypeStruct((T, E), jnp.float32),
        in_specs=[
            pl.BlockSpec((T, H), lambda: (0, 0)),
            pl.BlockSpec((E, H), lambda: (0, 0)),
        ],
        out_specs=pl.BlockSpec((T, E), lambda: (0, 0)),
    )(hidden_states, w_gate)

    x_bf = hidden_states.astype(jnp.bfloat16)
    w1_bf = w1.astype(jnp.bfloat16)
    w3_bf = w3.astype(jnp.bfloat16)
    w2_bf = w2.astype(jnp.bfloat16)

    out = pl.pallas_call(
        _moe_body,
        grid=(E, T_TILES),
        out_shape=jax.ShapeDtypeStruct((T, H), jnp.float32),
        in_specs=[
            pl.BlockSpec((TT, H), lambda e, t: (t, 0)),
            pl.BlockSpec((TT, E), lambda e, t: (t, 0)),
            pl.BlockSpec((1, F, H), lambda e, t: (e, 0, 0)),
            pl.BlockSpec((1, F, H), lambda e, t: (e, 0, 0)),
            pl.BlockSpec((1, H, F), lambda e, t: (e, 0, 0)),
        ],
        out_specs=pl.BlockSpec((T, H), lambda e, t: (0, 0)),
        compiler_params=pltpu.CompilerParams(
            dimension_semantics=("arbitrary", "arbitrary"),
        ),
    )(x_bf, combine, w1_bf, w3_bf, w2_bf)
    return out
